# initial kernel scaffold (unmeasured)
import jax
import jax.numpy as jnp
from jax import lax
from jax.experimental import pallas as pl
from jax.experimental.pallas import tpu as pltpu

N_DEV = 32
B_LOC = 512
D = 256
F_LOC = 512


def kernel(x, Win0, Wout0, Win1, Wout1, Win2, Wout2):
    def body(
        x_ref, win0_ref, wout0_ref, win1_ref, wout1_ref, win2_ref, wout2_ref,
        out_ref,
        wall_ref, wsend_sems, wrecv_sems, osend_sems, orecv_sems,
    ):
        my = lax.axis_index("i")
        left = lax.rem(my + N_DEV - 1, N_DEV)
        right = lax.rem(my + 1, N_DEV)

        for l, w in enumerate((win0_ref, win1_ref, win2_ref)):
            wall_ref[my, l] = jnp.transpose(w[...]).astype(jnp.bfloat16)
        for l, w in enumerate((wout0_ref, wout1_ref, wout2_ref)):
            wall_ref[my, 3 + l] = w[...].astype(jnp.bfloat16)

        barrier = pltpu.get_barrier_semaphore()
        for nbr in (left, right):
            pl.semaphore_signal(
                barrier, inc=1,
                device_id=(nbr,), device_id_type=pl.DeviceIdType.MESH,
            )
        pl.semaphore_wait(barrier, 2)

        for h in range(N_DEV - 1):
            so = lax.rem(my + N_DEV - h, N_DEV)
            rdma = pltpu.make_async_remote_copy(
                src_ref=wall_ref.at[so],
                dst_ref=wall_ref.at[so],
                send_sem=wsend_sems.at[h],
                recv_sem=wrecv_sems.at[h],
                device_id=(right,),
                device_id_type=pl.DeviceIdType.MESH,
            )
            rdma.start()
            rdma.wait()

        xb = x_ref[...].astype(jnp.bfloat16)
        for l in range(3):
            def step(j, acc, _l=l, _xb=xb):
                win_t = wall_ref[j, _l]
                wout = wall_ref[j, 3 + _l]
                hdn = lax.dot_general(
                    _xb, win_t, (((1,), (1,)), ((), ())),
                    preferred_element_type=jnp.float32,
                )
                hdn = jnp.maximum(hdn, 0.0).astype(jnp.bfloat16)
                part = lax.dot_general(
                    hdn, wout, (((1,), (0,)), ((), ())),
                    preferred_element_type=jnp.float32,
                )
                return acc + part

            acc = lax.fori_loop(
                0, N_DEV, step, jnp.zeros((B_LOC, D), jnp.float32)
            )
            xb = acc.astype(jnp.bfloat16)

        out_ref[pl.ds(my * B_LOC, B_LOC)] = xb

        for h in range(N_DEV - 1):
            so = lax.rem(my + N_DEV - h, N_DEV)
            rdma = pltpu.make_async_remote_copy(
                src_ref=out_ref.at[pl.ds(so * B_LOC, B_LOC)],
                dst_ref=out_ref.at[pl.ds(so * B_LOC, B_LOC)],
                send_sem=osend_sems.at[h],
                recv_sem=orecv_sems.at[h],
                device_id=(right,),
                device_id_type=pl.DeviceIdType.MESH,
            )
            rdma.start()
            rdma.wait()

    return pl.pallas_call(
        body,
        out_shape=jax.ShapeDtypeStruct((N_DEV * B_LOC, D), jnp.bfloat16),
        in_specs=[pl.BlockSpec(memory_space=pltpu.VMEM)] * 7,
        out_specs=pl.BlockSpec(memory_space=pltpu.VMEM),
        scratch_shapes=[
            pltpu.VMEM((N_DEV, 6, F_LOC, D), jnp.bfloat16),
            pltpu.SemaphoreType.DMA((N_DEV - 1,)),
            pltpu.SemaphoreType.DMA((N_DEV - 1,)),
            pltpu.SemaphoreType.DMA((N_DEV - 1,)),
            pltpu.SemaphoreType.DMA((N_DEV - 1,)),
        ],
        compiler_params=pltpu.CompilerParams(collective_id=0),
    )(x, Win0, Wout0, Win1, Wout1, Win2, Wout2)


# baseline (device time: 796655 ns/iter reference)
import jax
import jax.numpy as jnp
from jax import lax
from jax.experimental import pallas as pl
from jax.experimental.pallas import tpu as pltpu

N_DEV = 32
B_LOC = 512
D = 256
F_LOC = 512


def _pos_to_logical(p):
    x = jnp.where(p >= 16, 1, 0)
    q = jnp.where(x == 1, 31 - p, p)
    z = q // 4
    t = q - 4 * z
    y = jnp.where(z % 2 == 0, t, 3 - t)
    xb = jnp.where(y % 2 == 0, x, 1 - x)
    return 8 * z + 2 * y + xb


def _logical_to_pos(l):
    z = l // 8
    r = l - 8 * z
    y = r // 2
    xb = r - 2 * y
    x = jnp.where(y % 2 == 0, xb, 1 - xb)
    q = 4 * z + jnp.where(z % 2 == 0, y, 3 - y)
    return jnp.where(x == 0, q, 31 - q)


def kernel(x, Win0, Wout0, Win1, Wout1, Win2, Wout2):
    def body(
        x_ref, win0_ref, wout0_ref, win1_ref, wout1_ref, win2_ref, wout2_ref,
        out_ref,
        wall_ref, wsend_sems, wrecv_sems, osend_sems, orecv_sems,
    ):
        my = lax.axis_index("i")
        p = _logical_to_pos(my)
        right = _pos_to_logical(lax.rem(p + 1, N_DEV))
        left = _pos_to_logical(lax.rem(p + N_DEV - 1, N_DEV))

        for l, w in enumerate((win0_ref, win1_ref, win2_ref)):
            wall_ref[my, l] = jnp.transpose(w[...]).astype(jnp.bfloat16)
        for l, w in enumerate((wout0_ref, wout1_ref, wout2_ref)):
            wall_ref[my, 3 + l] = w[...].astype(jnp.bfloat16)

        barrier = pltpu.get_barrier_semaphore()
        for nbr in (left, right):
            pl.semaphore_signal(
                barrier, inc=1,
                device_id=(nbr,), device_id_type=pl.DeviceIdType.MESH,
            )
        pl.semaphore_wait(barrier, 2)

        for h in range(N_DEV - 1):
            so = _pos_to_logical(lax.rem(p + N_DEV - h, N_DEV))
            rdma = pltpu.make_async_remote_copy(
                src_ref=wall_ref.at[so],
                dst_ref=wall_ref.at[so],
                send_sem=wsend_sems.at[h],
                recv_sem=wrecv_sems.at[h],
                device_id=(right,),
                device_id_type=pl.DeviceIdType.MESH,
            )
            rdma.start()
            rdma.wait()

        xb = x_ref[...].astype(jnp.bfloat16)
        for l in range(3):
            def step(j, acc, _l=l, _xb=xb):
                win_t = wall_ref[j, _l]
                wout = wall_ref[j, 3 + _l]
                hdn = lax.dot_general(
                    _xb, win_t, (((1,), (1,)), ((), ())),
                    preferred_element_type=jnp.float32,
                )
                hdn = jnp.maximum(hdn, 0.0).astype(jnp.bfloat16)
                part = lax.dot_general(
                    hdn, wout, (((1,), (0,)), ((), ())),
                    preferred_element_type=jnp.float32,
                )
                return acc + part

            acc = lax.fori_loop(
                0, N_DEV, step, jnp.zeros((B_LOC, D), jnp.float32)
            )
            xb = acc.astype(jnp.bfloat16)

        out_ref[pl.ds(my * B_LOC, B_LOC)] = xb

        for h in range(N_DEV - 1):
            so = _pos_to_logical(lax.rem(p + N_DEV - h, N_DEV))
            rdma = pltpu.make_async_remote_copy(
                src_ref=out_ref.at[pl.ds(so * B_LOC, B_LOC)],
                dst_ref=out_ref.at[pl.ds(so * B_LOC, B_LOC)],
                send_sem=osend_sems.at[h],
                recv_sem=orecv_sems.at[h],
                device_id=(right,),
                device_id_type=pl.DeviceIdType.MESH,
            )
            rdma.start()
            rdma.wait()

    return pl.pallas_call(
        body,
        out_shape=jax.ShapeDtypeStruct((N_DEV * B_LOC, D), jnp.bfloat16),
        in_specs=[pl.BlockSpec(memory_space=pltpu.VMEM)] * 7,
        out_specs=pl.BlockSpec(memory_space=pltpu.VMEM),
        scratch_shapes=[
            pltpu.VMEM((N_DEV, 6, F_LOC, D), jnp.bfloat16),
            pltpu.SemaphoreType.DMA((N_DEV - 1,)),
            pltpu.SemaphoreType.DMA((N_DEV - 1,)),
            pltpu.SemaphoreType.DMA((N_DEV - 1,)),
            pltpu.SemaphoreType.DMA((N_DEV - 1,)),
        ],
        compiler_params=pltpu.CompilerParams(
            collective_id=0,
            vmem_limit_bytes=100 * 1024 * 1024,
        ),
    )(x, Win0, Wout0, Win1, Wout1, Win2, Wout2)
